# Initial kernel scaffold; baseline (speedup 1.0000x reference)
import jax
import jax.numpy as jnp
from jax.experimental import pallas as pl


def kernel(cls_out_0, cls_out_1, cls_out_2, cls_out_3, cls_out_4, box_out_0, box_out_1, box_out_2, box_out_3, box_out_4, gt_boxes, gt_classes):
    raise NotImplementedError("write your pallas kernel here")


# speedup vs baseline: 4.9196x; 4.9196x over previous
import numpy as np
import jax
import jax.numpy as jnp
from jax.experimental import pallas as pl

_IMAGE_SIZE = 512
_MIN_LEVEL = 3
_NUM_LEVELS = 5
_NUM_CLASSES = 90
_NUM_SCALES = 3
_ASPECTS = [(1.0, 1.0), (1.4, 0.7), (0.7, 1.4)]
_ANCHOR_SCALE = 4.0
_BATCH = 4
_NUM_ANCHORS = 49104
_BLOCK_A = 4464


def _anchor_aux_np():
    boxes_all = []
    for i in range(_NUM_LEVELS):
        stride = 2 ** (_MIN_LEVEL + i)
        per = []
        for octave in range(_NUM_SCALES):
            scale = 2.0 ** (octave / float(_NUM_SCALES))
            for (arh, arw) in _ASPECTS:
                base = _ANCHOR_SCALE * stride * scale
                hh = base * arh / 2.0
                hw = base * arw / 2.0
                c = np.arange(stride / 2.0, _IMAGE_SIZE, stride, dtype=np.float32)
                yv, xv = np.meshgrid(c, c, indexing='ij')
                per.append(
                    np.stack([yv - hh, xv - hw, yv + hh, xv + hw], axis=-1).reshape(-1, 4))
        boxes_all.append(np.stack(per, axis=1).reshape(-1, 4))
    a = np.concatenate(boxes_all, axis=0).astype(np.float32)
    aux = np.zeros((a.shape[0], 8), np.float32)
    aux[:, 0:4] = a
    aux[:, 4] = (a[:, 0] + a[:, 2]) / 2.0
    aux[:, 5] = (a[:, 1] + a[:, 3]) / 2.0
    aux[:, 6] = a[:, 2] - a[:, 0]
    aux[:, 7] = a[:, 3] - a[:, 1]
    return aux


_ANCHOR_AUX = _anchor_aux_np()


def _loss_kernel(cls_ref, box_ref, aux_ref, gt_ref, out_ref):
    b = pl.program_id(0)
    j = pl.program_id(1)

    aux = aux_ref[...]
    ay0 = aux[:, 0:1]
    ax0 = aux[:, 1:2]
    ay1 = aux[:, 2:3]
    ax1 = aux[:, 3:4]
    acy = aux[:, 4:5]
    acx = aux[:, 5:6]
    ah = aux[:, 6:7]
    aw = aux[:, 7:8]

    g = gt_ref[0]
    gy0 = g[0:1, :]
    gx0 = g[1:2, :]
    gy1 = g[2:3, :]
    gx1 = g[3:4, :]
    gcls = g[4:5, :]
    gcy = (gy0 + gy1) / 2.0
    gcx = (gx0 + gx1) / 2.0
    gh = gy1 - gy0
    gw = gx1 - gx0

    iy = jnp.maximum(0.0, jnp.minimum(ay1, gy1) - jnp.maximum(ay0, gy0))
    ix = jnp.maximum(0.0, jnp.minimum(ax1, gx1) - jnp.maximum(ax0, gx0))
    inter = iy * ix
    aa = (ay1 - ay0) * (ax1 - ax0)
    ga = (gy1 - gy0) * (gx1 - gx0)
    iou = inter / (aa + ga - inter + 1e-8)

    best_iou = jnp.max(iou, axis=1, keepdims=True)
    gidx = jax.lax.broadcasted_iota(jnp.int32, iou.shape, 1)
    bidx = jnp.min(jnp.where(iou == best_iou, gidx, 99), axis=1, keepdims=True)
    m = jnp.where(gidx == bidx, 1.0, 0.0)

    mcls = jnp.sum(m * gcls, axis=1, keepdims=True)
    mgcy = jnp.sum(m * gcy, axis=1, keepdims=True)
    mgcx = jnp.sum(m * gcx, axis=1, keepdims=True)
    mgh = jnp.sum(m * gh, axis=1, keepdims=True)
    mgw = jnp.sum(m * gw, axis=1, keepdims=True)
    matched = jnp.where(best_iou >= 0.5, 1.0, 0.0)

    l = cls_ref[0]
    cidx = jax.lax.broadcasted_iota(jnp.int32, l.shape, 1).astype(jnp.float32)
    t = jnp.where(cidx == mcls, 1.0, 0.0) * matched
    ce = jnp.maximum(l, 0.0) - l * t + jnp.log1p(jnp.exp(-jnp.abs(l)))
    p = jax.nn.sigmoid(l)
    pt = t * p + (1.0 - t) * (1.0 - p)
    at = t * 0.25 + (1.0 - t) * 0.75
    omp = 1.0 - pt
    focal = at * (omp * jnp.sqrt(omp)) * ce
    cls_sum = jnp.sum(focal)

    bx = box_ref[0]
    bt0 = (mgcy - acy) / ah
    bt1 = (mgcx - acx) / aw
    bt2 = jnp.log(mgh / ah)
    bt3 = jnp.log(mgw / aw)
    d = 0.1
    hsum = jnp.zeros_like(matched)
    for r, btr in enumerate((bt0, bt1, bt2, bt3)):
        err = bx[:, r:r + 1] - btr
        ae = jnp.abs(err)
        hsum = hsum + jnp.where(ae <= d, 0.5 * err * err, d * (ae - 0.5 * d))
    box_sum = jnp.sum(hsum * matched)
    pos_sum = jnp.sum(matched)

    lane = jax.lax.broadcasted_iota(jnp.int32, (1, 128), 1)
    contrib = (jnp.where(lane == 0, cls_sum, 0.0)
               + jnp.where(lane == 1, box_sum, 0.0)
               + jnp.where(lane == 2, pos_sum, 0.0))

    first = jnp.logical_and(b == 0, j == 0)

    @pl.when(first)
    def _():
        out_ref[...] = contrib

    @pl.when(jnp.logical_not(first))
    def _():
        out_ref[...] = out_ref[...] + contrib


def kernel(cls_out_0, cls_out_1, cls_out_2, cls_out_3, cls_out_4,
           box_out_0, box_out_1, box_out_2, box_out_3, box_out_4,
           gt_boxes, gt_classes):
    b = cls_out_0.shape[0]
    cls_all = jnp.concatenate(
        [o.transpose(0, 2, 3, 1).reshape(b, -1, _NUM_CLASSES)
         for o in (cls_out_0, cls_out_1, cls_out_2, cls_out_3, cls_out_4)], axis=1)
    box_all = jnp.concatenate(
        [o.transpose(0, 2, 3, 1).reshape(b, -1, 4)
         for o in (box_out_0, box_out_1, box_out_2, box_out_3, box_out_4)], axis=1)

    cy = gt_boxes[..., 0] * _IMAGE_SIZE
    cx = gt_boxes[..., 1] * _IMAGE_SIZE
    h = gt_boxes[..., 2] * 100.0 + 10.0
    w = gt_boxes[..., 3] * 100.0 + 10.0
    gt_rows = jnp.stack([cy - h / 2.0, cx - w / 2.0, cy + h / 2.0, cx + w / 2.0,
                         gt_classes.astype(jnp.float32)], axis=1)
    gt_aux = jnp.concatenate(
        [gt_rows, jnp.zeros((b, 3, gt_rows.shape[2]), jnp.float32)], axis=1)

    aux = jnp.asarray(_ANCHOR_AUX)

    grid = (b, _NUM_ANCHORS // _BLOCK_A)
    sums = pl.pallas_call(
        _loss_kernel,
        grid=grid,
        in_specs=[
            pl.BlockSpec((1, _BLOCK_A, _NUM_CLASSES), lambda bb, jj: (bb, jj, 0)),
            pl.BlockSpec((1, _BLOCK_A, 4), lambda bb, jj: (bb, jj, 0)),
            pl.BlockSpec((_BLOCK_A, 8), lambda bb, jj: (jj, 0)),
            pl.BlockSpec((1, 8, 32), lambda bb, jj: (bb, 0, 0)),
        ],
        out_specs=pl.BlockSpec((1, 128), lambda bb, jj: (0, 0)),
        out_shape=jax.ShapeDtypeStruct((1, 128), jnp.float32),
    )(cls_all, box_all, aux, gt_aux)

    s = sums[0]
    num_pos = s[2] + 1.0
    class_loss = s[0] / num_pos
    box_loss = s[1] / num_pos / 4.0
    loss = class_loss + 50.0 * box_loss
    return jnp.stack([loss, class_loss, box_loss])


# speedup vs baseline: 8.7448x; 1.7776x over previous
import numpy as np
import jax
import jax.numpy as jnp
from jax.experimental import pallas as pl

_IMAGE_SIZE = 512
_MIN_LEVEL = 3
_NUM_LEVELS = 5
_NUM_CLASSES = 90
_NUM_SCALES = 3
_ASPECTS = [(1.0, 1.0), (1.4, 0.7), (0.7, 1.4)]
_ANCHOR_SCALE = 4.0
_FEAT_HW = [64, 32, 16, 8, 4]


def _anchor_aux_np():
    out = []
    for i in range(_NUM_LEVELS):
        stride = 2 ** (_MIN_LEVEL + i)
        per = []
        for octave in range(_NUM_SCALES):
            scale = 2.0 ** (octave / float(_NUM_SCALES))
            for (arh, arw) in _ASPECTS:
                base = _ANCHOR_SCALE * stride * scale
                hh = base * arh / 2.0
                hw = base * arw / 2.0
                c = np.arange(stride / 2.0, _IMAGE_SIZE, stride, dtype=np.float32)
                yv, xv = np.meshgrid(c, c, indexing='ij')
                per.append(
                    np.stack([yv - hh, xv - hw, yv + hh, xv + hw], axis=-1).reshape(-1, 4))
        a = np.stack(per, axis=1).reshape(-1, 4).astype(np.float32)
        hwn = _FEAT_HW[i] * _FEAT_HW[i]
        aux = np.zeros((hwn * 9, 8), np.float32)
        aux[:, 0:4] = a
        aux[:, 4] = (a[:, 0] + a[:, 2]) / 2.0
        aux[:, 5] = (a[:, 1] + a[:, 3]) / 2.0
        aux[:, 6] = a[:, 2] - a[:, 0]
        aux[:, 7] = a[:, 3] - a[:, 1]
        out.append(np.ascontiguousarray(
            aux.reshape(hwn, 9, 8).transpose(1, 2, 0)))
    return out


_ANCHOR_AUX = _anchor_aux_np()


def _lvl_kernel(cls_ref, box_ref, aux_ref, gt_ref, out_ref):
    b = pl.program_id(0)
    a = pl.program_id(1)

    aux = aux_ref[0]
    ay0 = aux[0:1, :]
    ax0 = aux[1:2, :]
    ay1 = aux[2:3, :]
    ax1 = aux[3:4, :]
    acy = aux[4:5, :]
    acx = aux[5:6, :]
    ah = aux[6:7, :]
    aw = aux[7:8, :]

    g = gt_ref[0]
    gy0 = g[:, 0:1]
    gx0 = g[:, 1:2]
    gy1 = g[:, 2:3]
    gx1 = g[:, 3:4]
    gcls = g[:, 4:5]
    gcy = (gy0 + gy1) / 2.0
    gcx = (gx0 + gx1) / 2.0
    gh = gy1 - gy0
    gw = gx1 - gx0

    iy = jnp.maximum(0.0, jnp.minimum(ay1, gy1) - jnp.maximum(ay0, gy0))
    ix = jnp.maximum(0.0, jnp.minimum(ax1, gx1) - jnp.maximum(ax0, gx0))
    inter = iy * ix
    aa = (ay1 - ay0) * (ax1 - ax0)
    ga = (gy1 - gy0) * (gx1 - gx0)
    iou = inter / (aa + ga - inter + 1e-8)

    best = jnp.max(iou, axis=0, keepdims=True)
    gidx = jax.lax.broadcasted_iota(jnp.int32, iou.shape, 0)
    bidx = jnp.min(jnp.where(iou == best, gidx, 99), axis=0, keepdims=True)
    m = jnp.where(gidx == bidx, 1.0, 0.0)

    mcls = jnp.sum(m * gcls, axis=0, keepdims=True)
    mgcy = jnp.sum(m * gcy, axis=0, keepdims=True)
    mgcx = jnp.sum(m * gcx, axis=0, keepdims=True)
    mgh = jnp.sum(m * gh, axis=0, keepdims=True)
    mgw = jnp.sum(m * gw, axis=0, keepdims=True)
    matched = jnp.where(best >= 0.5, 1.0, 0.0)

    l = cls_ref[0, 0]
    cidx = jax.lax.broadcasted_iota(jnp.int32, l.shape, 0).astype(jnp.float32)
    t = jnp.where(cidx == mcls, 1.0, 0.0) * matched
    ce = jnp.maximum(l, 0.0) - l * t + jnp.log1p(jnp.exp(-jnp.abs(l)))
    p = jax.nn.sigmoid(l)
    pt = t * p + (1.0 - t) * (1.0 - p)
    at = t * 0.25 + (1.0 - t) * 0.75
    omp = 1.0 - pt
    focal = at * (omp * jnp.sqrt(omp)) * ce
    cls_sum = jnp.sum(focal)

    bx = box_ref[0, 0]
    bt0 = (mgcy - acy) / ah
    bt1 = (mgcx - acx) / aw
    bt2 = jnp.log(mgh / ah)
    bt3 = jnp.log(mgw / aw)
    d = 0.1
    hsum = jnp.zeros_like(matched)
    for r, btr in enumerate((bt0, bt1, bt2, bt3)):
        err = bx[r:r + 1, :] - btr
        ae = jnp.abs(err)
        hsum = hsum + jnp.where(ae <= d, 0.5 * err * err, d * (ae - 0.5 * d))
    box_sum = jnp.sum(hsum * matched)
    pos_sum = jnp.sum(matched)

    lane = jax.lax.broadcasted_iota(jnp.int32, (1, 128), 1)
    contrib = (jnp.where(lane == 0, cls_sum, 0.0)
               + jnp.where(lane == 1, box_sum, 0.0)
               + jnp.where(lane == 2, pos_sum, 0.0))

    first = jnp.logical_and(b == 0, a == 0)

    @pl.when(first)
    def _():
        out_ref[...] = contrib

    @pl.when(jnp.logical_not(first))
    def _():
        out_ref[...] = out_ref[...] + contrib


def _level_sums(cls_o, box_o, aux, gt_aux):
    b = cls_o.shape[0]
    hwn = cls_o.shape[2] * cls_o.shape[3]
    cls_r = cls_o.reshape(b, 9, _NUM_CLASSES, hwn)
    box_r = box_o.reshape(b, 9, 4, hwn)
    return pl.pallas_call(
        _lvl_kernel,
        grid=(b, 9),
        in_specs=[
            pl.BlockSpec((1, 1, _NUM_CLASSES, hwn), lambda bb, aa: (bb, aa, 0, 0)),
            pl.BlockSpec((1, 1, 4, hwn), lambda bb, aa: (bb, aa, 0, 0)),
            pl.BlockSpec((1, 8, hwn), lambda bb, aa: (aa, 0, 0)),
            pl.BlockSpec((1, 32, 8), lambda bb, aa: (bb, 0, 0)),
        ],
        out_specs=pl.BlockSpec((1, 128), lambda bb, aa: (0, 0)),
        out_shape=jax.ShapeDtypeStruct((1, 128), jnp.float32),
    )(cls_r, box_r, aux, gt_aux)


def kernel(cls_out_0, cls_out_1, cls_out_2, cls_out_3, cls_out_4,
           box_out_0, box_out_1, box_out_2, box_out_3, box_out_4,
           gt_boxes, gt_classes):
    b = cls_out_0.shape[0]

    cy = gt_boxes[..., 0] * _IMAGE_SIZE
    cx = gt_boxes[..., 1] * _IMAGE_SIZE
    h = gt_boxes[..., 2] * 100.0 + 10.0
    w = gt_boxes[..., 3] * 100.0 + 10.0
    gt_aux = jnp.stack(
        [cy - h / 2.0, cx - w / 2.0, cy + h / 2.0, cx + w / 2.0,
         gt_classes.astype(jnp.float32),
         jnp.zeros_like(cy), jnp.zeros_like(cy), jnp.zeros_like(cy)], axis=2)

    cls_outs = (cls_out_0, cls_out_1, cls_out_2, cls_out_3, cls_out_4)
    box_outs = (box_out_0, box_out_1, box_out_2, box_out_3, box_out_4)
    total = jnp.zeros((128,), jnp.float32)
    for i in range(_NUM_LEVELS):
        total = total + _level_sums(
            cls_outs[i], box_outs[i], jnp.asarray(_ANCHOR_AUX[i]), gt_aux)[0]

    num_pos = total[2] + 1.0
    class_loss = total[0] / num_pos
    box_loss = total[1] / num_pos / 4.0
    loss = class_loss + 50.0 * box_loss
    return jnp.stack([loss, class_loss, box_loss])


# speedup vs baseline: 8.9401x; 1.0223x over previous
import numpy as np
import jax
import jax.numpy as jnp
from jax.experimental import pallas as pl

_IMAGE_SIZE = 512
_MIN_LEVEL = 3
_NUM_LEVELS = 5
_NUM_CLASSES = 90
_NUM_SCALES = 3
_ASPECTS = [(1.0, 1.0), (1.4, 0.7), (0.7, 1.4)]
_ANCHOR_SCALE = 4.0
_FEAT_HW = [64, 32, 16, 8, 4]


def _anchor_aux_np():
    out = []
    for i in range(_NUM_LEVELS):
        stride = 2 ** (_MIN_LEVEL + i)
        per = []
        for octave in range(_NUM_SCALES):
            scale = 2.0 ** (octave / float(_NUM_SCALES))
            for (arh, arw) in _ASPECTS:
                base = _ANCHOR_SCALE * stride * scale
                hh = base * arh / 2.0
                hw = base * arw / 2.0
                c = np.arange(stride / 2.0, _IMAGE_SIZE, stride, dtype=np.float32)
                yv, xv = np.meshgrid(c, c, indexing='ij')
                per.append(
                    np.stack([yv - hh, xv - hw, yv + hh, xv + hw], axis=-1).reshape(-1, 4))
        a = np.stack(per, axis=1).reshape(-1, 4).astype(np.float32)
        hwn = _FEAT_HW[i] * _FEAT_HW[i]
        aux = np.zeros((hwn * 9, 8), np.float32)
        aux[:, 0:4] = a
        aux[:, 4] = (a[:, 0] + a[:, 2]) / 2.0
        aux[:, 5] = (a[:, 1] + a[:, 3]) / 2.0
        aux[:, 6] = a[:, 2] - a[:, 0]
        aux[:, 7] = a[:, 3] - a[:, 1]
        out.append(np.ascontiguousarray(
            aux.reshape(hwn, 9, 8).transpose(1, 2, 0)))
    return out


_ANCHOR_AUX = _anchor_aux_np()


def _process(l, bx, aux_a, g):
    ay0 = aux_a[0:1, :]
    ax0 = aux_a[1:2, :]
    ay1 = aux_a[2:3, :]
    ax1 = aux_a[3:4, :]
    acy = aux_a[4:5, :]
    acx = aux_a[5:6, :]
    ah = aux_a[6:7, :]
    aw = aux_a[7:8, :]

    gy0 = g[:, 0:1]
    gx0 = g[:, 1:2]
    gy1 = g[:, 2:3]
    gx1 = g[:, 3:4]
    gcls = g[:, 4:5]
    gcy = (gy0 + gy1) / 2.0
    gcx = (gx0 + gx1) / 2.0
    gh = gy1 - gy0
    gw = gx1 - gx0

    iy = jnp.maximum(0.0, jnp.minimum(ay1, gy1) - jnp.maximum(ay0, gy0))
    ix = jnp.maximum(0.0, jnp.minimum(ax1, gx1) - jnp.maximum(ax0, gx0))
    inter = iy * ix
    aa = (ay1 - ay0) * (ax1 - ax0)
    ga = (gy1 - gy0) * (gx1 - gx0)
    iou = inter / (aa + ga - inter + 1e-8)

    best = jnp.max(iou, axis=0, keepdims=True)
    gidx = jax.lax.broadcasted_iota(jnp.int32, iou.shape, 0)
    bidx = jnp.min(jnp.where(iou == best, gidx, 99), axis=0, keepdims=True)
    m = jnp.where(gidx == bidx, 1.0, 0.0)

    mcls = jnp.sum(m * gcls, axis=0, keepdims=True)
    mgcy = jnp.sum(m * gcy, axis=0, keepdims=True)
    mgcx = jnp.sum(m * gcx, axis=0, keepdims=True)
    mgh = jnp.sum(m * gh, axis=0, keepdims=True)
    mgw = jnp.sum(m * gw, axis=0, keepdims=True)
    matched = jnp.where(best >= 0.5, 1.0, 0.0)

    cidx = jax.lax.broadcasted_iota(jnp.int32, l.shape, 0).astype(jnp.float32)
    t = jnp.where(cidx == mcls, 1.0, 0.0) * matched
    ce = jnp.maximum(l, 0.0) - l * t + jnp.log1p(jnp.exp(-jnp.abs(l)))
    p = jax.nn.sigmoid(l)
    pt = t * p + (1.0 - t) * (1.0 - p)
    at = t * 0.25 + (1.0 - t) * 0.75
    omp = 1.0 - pt
    focal = at * (omp * jnp.sqrt(omp)) * ce
    cls_sum = jnp.sum(focal)

    bt0 = (mgcy - acy) / ah
    bt1 = (mgcx - acx) / aw
    bt2 = jnp.log(mgh / ah)
    bt3 = jnp.log(mgw / aw)
    d = 0.1
    hsum = jnp.zeros_like(matched)
    for r, btr in enumerate((bt0, bt1, bt2, bt3)):
        err = bx[r:r + 1, :] - btr
        ae = jnp.abs(err)
        hsum = hsum + jnp.where(ae <= d, 0.5 * err * err, d * (ae - 0.5 * d))
    box_sum = jnp.sum(hsum * matched)
    pos_sum = jnp.sum(matched)
    return cls_sum, box_sum, pos_sum


def _loss_kernel(*refs):
    cls_refs = refs[0:_NUM_LEVELS]
    box_refs = refs[_NUM_LEVELS:2 * _NUM_LEVELS]
    aux_refs = refs[2 * _NUM_LEVELS:3 * _NUM_LEVELS]
    gt_ref = refs[3 * _NUM_LEVELS]
    out_ref = refs[3 * _NUM_LEVELS + 1]

    b = pl.program_id(0)
    g = gt_ref[0]

    cls_sum = jnp.float32(0.0)
    box_sum = jnp.float32(0.0)
    pos_sum = jnp.float32(0.0)
    for i in range(_NUM_LEVELS):
        aux = aux_refs[i][...]
        for a in range(9):
            cs, bs, ps = _process(
                cls_refs[i][0, a], box_refs[i][0, a], aux[a], g)
            cls_sum += cs
            box_sum += bs
            pos_sum += ps

    lane = jax.lax.broadcasted_iota(jnp.int32, (1, 128), 1)
    contrib = (jnp.where(lane == 0, cls_sum, 0.0)
               + jnp.where(lane == 1, box_sum, 0.0)
               + jnp.where(lane == 2, pos_sum, 0.0))

    @pl.when(b == 0)
    def _():
        out_ref[...] = contrib

    @pl.when(b != 0)
    def _():
        out_ref[...] = out_ref[...] + contrib


def kernel(cls_out_0, cls_out_1, cls_out_2, cls_out_3, cls_out_4,
           box_out_0, box_out_1, box_out_2, box_out_3, box_out_4,
           gt_boxes, gt_classes):
    b = cls_out_0.shape[0]

    cy = gt_boxes[..., 0] * _IMAGE_SIZE
    cx = gt_boxes[..., 1] * _IMAGE_SIZE
    h = gt_boxes[..., 2] * 100.0 + 10.0
    w = gt_boxes[..., 3] * 100.0 + 10.0
    gt_aux = jnp.stack(
        [cy - h / 2.0, cx - w / 2.0, cy + h / 2.0, cx + w / 2.0,
         gt_classes.astype(jnp.float32),
         jnp.zeros_like(cy), jnp.zeros_like(cy), jnp.zeros_like(cy)], axis=2)

    cls_outs = (cls_out_0, cls_out_1, cls_out_2, cls_out_3, cls_out_4)
    box_outs = (box_out_0, box_out_1, box_out_2, box_out_3, box_out_4)
    hwn = [hw * hw for hw in _FEAT_HW]
    cls_r = [o.reshape(b, 9, _NUM_CLASSES, hwn[i]) for i, o in enumerate(cls_outs)]
    box_r = [o.reshape(b, 9, 4, hwn[i]) for i, o in enumerate(box_outs)]
    aux = [jnp.asarray(a) for a in _ANCHOR_AUX]

    def _cls_spec(i):
        return pl.BlockSpec((1, 9, _NUM_CLASSES, hwn[i]), lambda bb: (bb, 0, 0, 0))

    def _box_spec(i):
        return pl.BlockSpec((1, 9, 4, hwn[i]), lambda bb: (bb, 0, 0, 0))

    def _aux_spec(i):
        return pl.BlockSpec((9, 8, hwn[i]), lambda bb: (0, 0, 0))

    sums = pl.pallas_call(
        _loss_kernel,
        grid=(b,),
        in_specs=([_cls_spec(i) for i in range(_NUM_LEVELS)]
                  + [_box_spec(i) for i in range(_NUM_LEVELS)]
                  + [_aux_spec(i) for i in range(_NUM_LEVELS)]
                  + [pl.BlockSpec((1, 32, 8), lambda bb: (bb, 0, 0))]),
        out_specs=pl.BlockSpec((1, 128), lambda bb: (0, 0)),
        out_shape=jax.ShapeDtypeStruct((1, 128), jnp.float32),
    )(*cls_r, *box_r, *aux, gt_aux)

    s = sums[0]
    num_pos = s[2] + 1.0
    class_loss = s[0] / num_pos
    box_loss = s[1] / num_pos / 4.0
    loss = class_loss + 50.0 * box_loss
    return jnp.stack([loss, class_loss, box_loss])


# speedup vs baseline: 9.7560x; 1.0913x over previous
import numpy as np
import jax
import jax.numpy as jnp
from jax.experimental import pallas as pl

_IMAGE_SIZE = 512
_MIN_LEVEL = 3
_NUM_LEVELS = 5
_NUM_CLASSES = 90
_NUM_SCALES = 3
_ASPECTS = [(1.0, 1.0), (1.4, 0.7), (0.7, 1.4)]
_ANCHOR_SCALE = 4.0
_FEAT_HW = [64, 32, 16, 8, 4]


def _anchor_aux_np():
    out = []
    for i in range(_NUM_LEVELS):
        stride = 2 ** (_MIN_LEVEL + i)
        per = []
        for octave in range(_NUM_SCALES):
            scale = 2.0 ** (octave / float(_NUM_SCALES))
            for (arh, arw) in _ASPECTS:
                base = _ANCHOR_SCALE * stride * scale
                hh = base * arh / 2.0
                hw = base * arw / 2.0
                c = np.arange(stride / 2.0, _IMAGE_SIZE, stride, dtype=np.float32)
                yv, xv = np.meshgrid(c, c, indexing='ij')
                per.append(
                    np.stack([yv - hh, xv - hw, yv + hh, xv + hw], axis=-1).reshape(-1, 4))
        a = np.stack(per, axis=1).reshape(-1, 4).astype(np.float32)
        hwn = _FEAT_HW[i] * _FEAT_HW[i]
        aux = np.zeros((hwn * 9, 8), np.float32)
        aux[:, 0:4] = a
        aux[:, 4] = (a[:, 0] + a[:, 2]) / 2.0
        aux[:, 5] = (a[:, 1] + a[:, 3]) / 2.0
        aux[:, 6] = a[:, 2] - a[:, 0]
        aux[:, 7] = a[:, 3] - a[:, 1]
        out.append(np.ascontiguousarray(
            aux.reshape(hwn, 9, 8).transpose(1, 2, 0)))
    return out


_ANCHOR_AUX = _anchor_aux_np()


def _process(l, bx, aux_a, g):
    ay0 = aux_a[0:1, :]
    ax0 = aux_a[1:2, :]
    ay1 = aux_a[2:3, :]
    ax1 = aux_a[3:4, :]
    acy = aux_a[4:5, :]
    acx = aux_a[5:6, :]
    ah = aux_a[6:7, :]
    aw = aux_a[7:8, :]

    gy0 = g[:, 0:1]
    gx0 = g[:, 1:2]
    gy1 = g[:, 2:3]
    gx1 = g[:, 3:4]
    gcls = g[:, 4:5]
    gcy = (gy0 + gy1) / 2.0
    gcx = (gx0 + gx1) / 2.0
    gh = gy1 - gy0
    gw = gx1 - gx0

    iy = jnp.maximum(0.0, jnp.minimum(ay1, gy1) - jnp.maximum(ay0, gy0))
    ix = jnp.maximum(0.0, jnp.minimum(ax1, gx1) - jnp.maximum(ax0, gx0))
    inter = iy * ix
    aa = (ay1 - ay0) * (ax1 - ax0)
    ga = (gy1 - gy0) * (gx1 - gx0)
    iou = inter / (aa + ga - inter + 1e-8)

    best = jnp.max(iou, axis=0, keepdims=True)
    gidx = jax.lax.broadcasted_iota(jnp.int32, iou.shape, 0)
    bidx = jnp.min(jnp.where(iou == best, gidx, 99), axis=0, keepdims=True)
    m = jnp.where(gidx == bidx, 1.0, 0.0)

    mcls = jnp.sum(m * gcls, axis=0, keepdims=True)
    mgcy = jnp.sum(m * gcy, axis=0, keepdims=True)
    mgcx = jnp.sum(m * gcx, axis=0, keepdims=True)
    mgh = jnp.sum(m * gh, axis=0, keepdims=True)
    mgw = jnp.sum(m * gw, axis=0, keepdims=True)
    matched = jnp.where(best >= 0.5, 1.0, 0.0)

    e = jnp.exp(-jnp.abs(l))
    sp0 = jnp.maximum(l, 0.0) + jnp.log1p(e)
    r = 1.0 / (1.0 + e)
    p = jnp.where(l >= 0.0, r, 1.0 - r)
    f0 = 0.75 * (p * jnp.sqrt(p)) * sp0
    cls_sum0 = jnp.sum(f0)

    cidx = jax.lax.broadcasted_iota(jnp.int32, l.shape, 0).astype(jnp.float32)
    tsel = jnp.logical_and(cidx == mcls, matched > 0.0)
    lm = jnp.sum(jnp.where(tsel, l, 0.0), axis=0, keepdims=True)
    em = jnp.exp(-jnp.abs(lm))
    spm0 = jnp.maximum(lm, 0.0) + jnp.log1p(em)
    rm = 1.0 / (1.0 + em)
    pm = jnp.where(lm >= 0.0, rm, 1.0 - rm)
    f0m = 0.75 * (pm * jnp.sqrt(pm)) * spm0
    qm = 1.0 - pm
    f1m = 0.25 * (qm * jnp.sqrt(qm)) * (spm0 - lm)
    cls_sum = cls_sum0 + jnp.sum((f1m - f0m) * matched)

    bt0 = (mgcy - acy) / ah
    bt1 = (mgcx - acx) / aw
    bt2 = jnp.log(mgh / ah)
    bt3 = jnp.log(mgw / aw)
    d = 0.1
    hsum = jnp.zeros_like(matched)
    for r, btr in enumerate((bt0, bt1, bt2, bt3)):
        err = bx[r:r + 1, :] - btr
        ae = jnp.abs(err)
        hsum = hsum + jnp.where(ae <= d, 0.5 * err * err, d * (ae - 0.5 * d))
    box_sum = jnp.sum(hsum * matched)
    pos_sum = jnp.sum(matched)
    return cls_sum, box_sum, pos_sum


def _loss_kernel(*refs):
    cls_refs = refs[0:_NUM_LEVELS]
    box_refs = refs[_NUM_LEVELS:2 * _NUM_LEVELS]
    aux_refs = refs[2 * _NUM_LEVELS:3 * _NUM_LEVELS]
    gt_ref = refs[3 * _NUM_LEVELS]
    out_ref = refs[3 * _NUM_LEVELS + 1]

    b = pl.program_id(0)
    g = gt_ref[0]

    cls_sum = jnp.float32(0.0)
    box_sum = jnp.float32(0.0)
    pos_sum = jnp.float32(0.0)
    for i in range(_NUM_LEVELS):
        aux = aux_refs[i][...]
        for a in range(9):
            cs, bs, ps = _process(
                cls_refs[i][0, a], box_refs[i][0, a], aux[a], g)
            cls_sum += cs
            box_sum += bs
            pos_sum += ps

    lane = jax.lax.broadcasted_iota(jnp.int32, (1, 128), 1)
    contrib = (jnp.where(lane == 0, cls_sum, 0.0)
               + jnp.where(lane == 1, box_sum, 0.0)
               + jnp.where(lane == 2, pos_sum, 0.0))

    @pl.when(b == 0)
    def _():
        out_ref[...] = contrib

    @pl.when(b != 0)
    def _():
        out_ref[...] = out_ref[...] + contrib


def kernel(cls_out_0, cls_out_1, cls_out_2, cls_out_3, cls_out_4,
           box_out_0, box_out_1, box_out_2, box_out_3, box_out_4,
           gt_boxes, gt_classes):
    b = cls_out_0.shape[0]

    cy = gt_boxes[..., 0] * _IMAGE_SIZE
    cx = gt_boxes[..., 1] * _IMAGE_SIZE
    h = gt_boxes[..., 2] * 100.0 + 10.0
    w = gt_boxes[..., 3] * 100.0 + 10.0
    gt_aux = jnp.stack(
        [cy - h / 2.0, cx - w / 2.0, cy + h / 2.0, cx + w / 2.0,
         gt_classes.astype(jnp.float32),
         jnp.zeros_like(cy), jnp.zeros_like(cy), jnp.zeros_like(cy)], axis=2)

    cls_outs = (cls_out_0, cls_out_1, cls_out_2, cls_out_3, cls_out_4)
    box_outs = (box_out_0, box_out_1, box_out_2, box_out_3, box_out_4)
    hwn = [hw * hw for hw in _FEAT_HW]
    cls_r = [o.reshape(b, 9, _NUM_CLASSES, hwn[i]) for i, o in enumerate(cls_outs)]
    box_r = [o.reshape(b, 9, 4, hwn[i]) for i, o in enumerate(box_outs)]
    aux = [jnp.asarray(a) for a in _ANCHOR_AUX]

    def _cls_spec(i):
        return pl.BlockSpec((1, 9, _NUM_CLASSES, hwn[i]), lambda bb: (bb, 0, 0, 0))

    def _box_spec(i):
        return pl.BlockSpec((1, 9, 4, hwn[i]), lambda bb: (bb, 0, 0, 0))

    def _aux_spec(i):
        return pl.BlockSpec((9, 8, hwn[i]), lambda bb: (0, 0, 0))

    sums = pl.pallas_call(
        _loss_kernel,
        grid=(b,),
        in_specs=([_cls_spec(i) for i in range(_NUM_LEVELS)]
                  + [_box_spec(i) for i in range(_NUM_LEVELS)]
                  + [_aux_spec(i) for i in range(_NUM_LEVELS)]
                  + [pl.BlockSpec((1, 32, 8), lambda bb: (bb, 0, 0))]),
        out_specs=pl.BlockSpec((1, 128), lambda bb: (0, 0)),
        out_shape=jax.ShapeDtypeStruct((1, 128), jnp.float32),
    )(*cls_r, *box_r, *aux, gt_aux)

    s = sums[0]
    num_pos = s[2] + 1.0
    class_loss = s[0] / num_pos
    box_loss = s[1] / num_pos / 4.0
    loss = class_loss + 50.0 * box_loss
    return jnp.stack([loss, class_loss, box_loss])


# speedup vs baseline: 10.2029x; 1.0458x over previous
import numpy as np
import jax
import jax.numpy as jnp
from jax.experimental import pallas as pl

_IMAGE_SIZE = 512
_MIN_LEVEL = 3
_NUM_LEVELS = 5
_NUM_CLASSES = 90
_NUM_SCALES = 3
_ASPECTS = [(1.0, 1.0), (1.4, 0.7), (0.7, 1.4)]
_ANCHOR_SCALE = 4.0
_FEAT_HW = [64, 32, 16, 8, 4]


def _anchor_aux_np():
    out = []
    for i in range(_NUM_LEVELS):
        stride = 2 ** (_MIN_LEVEL + i)
        per = []
        for octave in range(_NUM_SCALES):
            scale = 2.0 ** (octave / float(_NUM_SCALES))
            for (arh, arw) in _ASPECTS:
                base = _ANCHOR_SCALE * stride * scale
                hh = base * arh / 2.0
                hw = base * arw / 2.0
                c = np.arange(stride / 2.0, _IMAGE_SIZE, stride, dtype=np.float32)
                yv, xv = np.meshgrid(c, c, indexing='ij')
                per.append(
                    np.stack([yv - hh, xv - hw, yv + hh, xv + hw], axis=-1).reshape(-1, 4))
        a = np.stack(per, axis=1).reshape(-1, 4).astype(np.float32)
        hwn = _FEAT_HW[i] * _FEAT_HW[i]
        aux = np.zeros((hwn * 9, 8), np.float32)
        aux[:, 0:4] = a
        aux[:, 4] = (a[:, 0] + a[:, 2]) / 2.0
        aux[:, 5] = (a[:, 1] + a[:, 3]) / 2.0
        aux[:, 6] = a[:, 2] - a[:, 0]
        aux[:, 7] = a[:, 3] - a[:, 1]
        out.append(np.ascontiguousarray(
            aux.reshape(hwn, 9, 8).transpose(1, 2, 0)))
    return out


_ANCHOR_AUX = _anchor_aux_np()


_CHUNK = 512


def _process(l, bx, aux_a, g):
    ay0 = aux_a[0:1, :]
    ax0 = aux_a[1:2, :]
    ay1 = aux_a[2:3, :]
    ax1 = aux_a[3:4, :]
    acy = aux_a[4:5, :]
    acx = aux_a[5:6, :]
    ah = aux_a[6:7, :]
    aw = aux_a[7:8, :]

    gy0 = g[:, 0:1]
    gx0 = g[:, 1:2]
    gy1 = g[:, 2:3]
    gx1 = g[:, 3:4]
    gcls = g[:, 4:5]
    gcy = (gy0 + gy1) / 2.0
    gcx = (gx0 + gx1) / 2.0
    gh = gy1 - gy0
    gw = gx1 - gx0

    iy = jnp.maximum(0.0, jnp.minimum(ay1, gy1) - jnp.maximum(ay0, gy0))
    ix = jnp.maximum(0.0, jnp.minimum(ax1, gx1) - jnp.maximum(ax0, gx0))
    inter = iy * ix
    aa = (ay1 - ay0) * (ax1 - ax0)
    ga = (gy1 - gy0) * (gx1 - gx0)
    iou = inter / (aa + ga - inter + 1e-8)

    best = jnp.max(iou, axis=0, keepdims=True)
    gidx = jax.lax.broadcasted_iota(jnp.int32, iou.shape, 0)
    bidx = jnp.min(jnp.where(iou == best, gidx, 99), axis=0, keepdims=True)
    m = jnp.where(gidx == bidx, 1.0, 0.0)

    mcls = jnp.sum(m * gcls, axis=0, keepdims=True)
    mgcy = jnp.sum(m * gcy, axis=0, keepdims=True)
    mgcx = jnp.sum(m * gcx, axis=0, keepdims=True)
    mgh = jnp.sum(m * gh, axis=0, keepdims=True)
    mgw = jnp.sum(m * gw, axis=0, keepdims=True)
    matched = jnp.where(best >= 0.5, 1.0, 0.0)

    e = jnp.exp(-jnp.abs(l))
    sp0 = jnp.maximum(l, 0.0) + jnp.log1p(e)
    r = 1.0 / (1.0 + e)
    p = jnp.where(l >= 0.0, r, 1.0 - r)
    f0 = 0.75 * (p * jnp.sqrt(p)) * sp0
    cls_sum0 = jnp.sum(f0)

    cidx = jax.lax.broadcasted_iota(jnp.int32, l.shape, 0).astype(jnp.float32)
    tsel = jnp.logical_and(cidx == mcls, matched > 0.0)
    lm = jnp.sum(jnp.where(tsel, l, 0.0), axis=0, keepdims=True)
    em = jnp.exp(-jnp.abs(lm))
    spm0 = jnp.maximum(lm, 0.0) + jnp.log1p(em)
    rm = 1.0 / (1.0 + em)
    pm = jnp.where(lm >= 0.0, rm, 1.0 - rm)
    f0m = 0.75 * (pm * jnp.sqrt(pm)) * spm0
    qm = 1.0 - pm
    f1m = 0.25 * (qm * jnp.sqrt(qm)) * (spm0 - lm)
    cls_sum = cls_sum0 + jnp.sum((f1m - f0m) * matched)

    bt0 = (mgcy - acy) / ah
    bt1 = (mgcx - acx) / aw
    bt2 = jnp.log(mgh / ah)
    bt3 = jnp.log(mgw / aw)
    d = 0.1
    hsum = jnp.zeros_like(matched)
    for r, btr in enumerate((bt0, bt1, bt2, bt3)):
        err = bx[r:r + 1, :] - btr
        ae = jnp.abs(err)
        hsum = hsum + jnp.where(ae <= d, 0.5 * err * err, d * (ae - 0.5 * d))
    box_sum = jnp.sum(hsum * matched)
    pos_sum = jnp.sum(matched)
    return cls_sum, box_sum, pos_sum


def _loss_kernel(*refs):
    cls_refs = refs[0:_NUM_LEVELS]
    box_refs = refs[_NUM_LEVELS:2 * _NUM_LEVELS]
    aux_refs = refs[2 * _NUM_LEVELS:3 * _NUM_LEVELS]
    gt_ref = refs[3 * _NUM_LEVELS]
    out_ref = refs[3 * _NUM_LEVELS + 1]

    b = pl.program_id(0)
    g = gt_ref[0]

    cls_sum = jnp.float32(0.0)
    box_sum = jnp.float32(0.0)
    pos_sum = jnp.float32(0.0)
    for i in range(_NUM_LEVELS):
        hwn = _FEAT_HW[i] * _FEAT_HW[i]
        ch = min(_CHUNK, hwn)
        for a in range(9):
            for c0 in range(0, hwn, ch):
                cs, bs, ps = _process(
                    cls_refs[i][0, a, :, c0:c0 + ch],
                    box_refs[i][0, a, :, c0:c0 + ch],
                    aux_refs[i][a, :, c0:c0 + ch], g)
                cls_sum += cs
                box_sum += bs
                pos_sum += ps

    lane = jax.lax.broadcasted_iota(jnp.int32, (1, 128), 1)
    contrib = (jnp.where(lane == 0, cls_sum, 0.0)
               + jnp.where(lane == 1, box_sum, 0.0)
               + jnp.where(lane == 2, pos_sum, 0.0))

    @pl.when(b == 0)
    def _():
        out_ref[...] = contrib

    @pl.when(b != 0)
    def _():
        out_ref[...] = out_ref[...] + contrib


def kernel(cls_out_0, cls_out_1, cls_out_2, cls_out_3, cls_out_4,
           box_out_0, box_out_1, box_out_2, box_out_3, box_out_4,
           gt_boxes, gt_classes):
    b = cls_out_0.shape[0]

    cy = gt_boxes[..., 0] * _IMAGE_SIZE
    cx = gt_boxes[..., 1] * _IMAGE_SIZE
    h = gt_boxes[..., 2] * 100.0 + 10.0
    w = gt_boxes[..., 3] * 100.0 + 10.0
    gt_aux = jnp.stack(
        [cy - h / 2.0, cx - w / 2.0, cy + h / 2.0, cx + w / 2.0,
         gt_classes.astype(jnp.float32),
         jnp.zeros_like(cy), jnp.zeros_like(cy), jnp.zeros_like(cy)], axis=2)

    cls_outs = (cls_out_0, cls_out_1, cls_out_2, cls_out_3, cls_out_4)
    box_outs = (box_out_0, box_out_1, box_out_2, box_out_3, box_out_4)
    hwn = [hw * hw for hw in _FEAT_HW]
    cls_r = [o.reshape(b, 9, _NUM_CLASSES, hwn[i]) for i, o in enumerate(cls_outs)]
    box_r = [o.reshape(b, 9, 4, hwn[i]) for i, o in enumerate(box_outs)]
    aux = [jnp.asarray(a) for a in _ANCHOR_AUX]

    def _cls_spec(i):
        return pl.BlockSpec((1, 9, _NUM_CLASSES, hwn[i]), lambda bb: (bb, 0, 0, 0))

    def _box_spec(i):
        return pl.BlockSpec((1, 9, 4, hwn[i]), lambda bb: (bb, 0, 0, 0))

    def _aux_spec(i):
        return pl.BlockSpec((9, 8, hwn[i]), lambda bb: (0, 0, 0))

    sums = pl.pallas_call(
        _loss_kernel,
        grid=(b,),
        in_specs=([_cls_spec(i) for i in range(_NUM_LEVELS)]
                  + [_box_spec(i) for i in range(_NUM_LEVELS)]
                  + [_aux_spec(i) for i in range(_NUM_LEVELS)]
                  + [pl.BlockSpec((1, 32, 8), lambda bb: (bb, 0, 0))]),
        out_specs=pl.BlockSpec((1, 128), lambda bb: (0, 0)),
        out_shape=jax.ShapeDtypeStruct((1, 128), jnp.float32),
    )(*cls_r, *box_r, *aux, gt_aux)

    s = sums[0]
    num_pos = s[2] + 1.0
    class_loss = s[0] / num_pos
    box_loss = s[1] / num_pos / 4.0
    loss = class_loss + 50.0 * box_loss
    return jnp.stack([loss, class_loss, box_loss])
